# baseline (speedup 1.0000x reference)
import jax
import jax.numpy as jnp
from jax import lax
from jax.experimental import pallas as pl
from jax.experimental.pallas import tpu as pltpu
from jax.experimental.pallas import tpu_sc as plsc

_BB = 128


def kernel(input_ids, weight):
    batch, seq = input_ids.shape
    n = batch * seq
    emb_dim = weight.shape[1]
    idx_t = input_ids.T.astype(jnp.int32)

    info = plsc.get_sparse_core_info()
    nw = info.num_cores * info.num_subcores

    mesh = plsc.VectorSubcoreMesh(
        core_axis_name="core", subcore_axis_name="subcore"
    )

    n_blocks = batch // _BB
    s_per_w = (seq * n_blocks) // nw

    @pl.kernel(
        out_type=jax.ShapeDtypeStruct((seq, emb_dim, batch), weight.dtype),
        mesh=mesh,
        scratch_types=[
            pltpu.VMEM((_BB,), jnp.int32),
            pltpu.VMEM((_BB, emb_dim), weight.dtype),
            pltpu.VMEM((emb_dim, _BB), weight.dtype),
            pltpu.SemaphoreType.DMA,
        ],
        compiler_params=pltpu.CompilerParams(
            use_tc_tiling_on_sc=False, needs_layout_passes=False
        ),
    )
    def gather_kernel(table_hbm, idx_hbm, out_hbm, idx_v, rows_v, rows_t_v, sem):
        wid = lax.axis_index("subcore") * info.num_cores + lax.axis_index(
            "core"
        )

        @pl.loop(0, s_per_w)
        def _(item):
            flat = wid * s_per_w + item
            s = flat // n_blocks
            bb = flat % n_blocks
            pltpu.sync_copy(idx_hbm.at[s, pl.ds(bb * _BB, _BB)], idx_v)
            pltpu.async_copy(table_hbm.at[idx_v], rows_v, sem).wait()
            @pl.loop(0, emb_dim)
            def _(c):
                cvec = jnp.full((16,), c, dtype=jnp.int32)

                @pl.loop(0, _BB, step=16)
                def _(b0):
                    lane = lax.iota(jnp.int32, 16) + b0
                    vals = plsc.load_gather(rows_v, [lane, cvec])
                    rows_t_v[c, pl.ds(b0, 16)] = vals

            pltpu.sync_copy(
                rows_t_v, out_hbm.at[s, :, pl.ds(bb * _BB, _BB)]
            )

    out_t = gather_kernel(weight, idx_t)
    return out_t.transpose(2, 0, 1)


# speedup vs baseline: 1.0967x; 1.0967x over previous
import jax
import jax.numpy as jnp
from jax import lax
from jax.experimental import pallas as pl
from jax.experimental.pallas import tpu as pltpu
from jax.experimental.pallas import tpu_sc as plsc

_BB = 128


def kernel(input_ids, weight):
    batch, seq = input_ids.shape
    emb_dim = weight.shape[1]
    idx_t = input_ids.T.astype(jnp.int32)

    info = plsc.get_sparse_core_info()
    nw = info.num_cores * info.num_subcores

    mesh = plsc.VectorSubcoreMesh(
        core_axis_name="core", subcore_axis_name="subcore"
    )

    n_blocks = batch // _BB
    items_per_w = (seq * n_blocks) // nw

    @pl.kernel(
        out_type=jax.ShapeDtypeStruct((seq, emb_dim, batch), weight.dtype),
        mesh=mesh,
        scratch_types=[
            pltpu.VMEM((2, _BB), jnp.int32),
            pltpu.VMEM((_BB, emb_dim), weight.dtype),
            pltpu.VMEM((_BB, emb_dim), weight.dtype),
            pltpu.VMEM((emb_dim, _BB), weight.dtype),
            pltpu.SemaphoreType.DMA,
            pltpu.SemaphoreType.DMA,
        ],
        compiler_params=pltpu.CompilerParams(
            use_tc_tiling_on_sc=False, needs_layout_passes=False
        ),
    )
    def gather_kernel(
        table_hbm, idx_hbm, out_hbm, idx_v, rows_a, rows_b, rows_t, sem_a,
        sem_b,
    ):
        wid = lax.axis_index("subcore") * info.num_cores + lax.axis_index(
            "core"
        )
        base = wid * items_per_w

        def fetch(item, idx_row, rows_v, sem):
            flat = base + item
            s = flat // n_blocks
            bb = flat % n_blocks
            pltpu.sync_copy(
                idx_hbm.at[s, pl.ds(bb * _BB, _BB)], idx_v.at[idx_row]
            )
            return pltpu.async_copy(table_hbm.at[idx_v.at[idx_row]], rows_v, sem)

        def emit(item, idx_row, rows_v, sem):
            flat = base + item
            s = flat // n_blocks
            bb = flat % n_blocks
            pltpu.make_async_copy(
                table_hbm.at[idx_v.at[idx_row]], rows_v, sem
            ).wait()
            iota16 = lax.iota(jnp.int32, 16)
            for c in range(emb_dim):
                cvec = jnp.full((16,), c, dtype=jnp.int32)
                for k in range(_BB // 16):
                    vals = plsc.load_gather(rows_v, [iota16 + k * 16, cvec])
                    rows_t[c, pl.ds(k * 16, 16)] = vals
            pltpu.sync_copy(rows_t, out_hbm.at[s, :, pl.ds(bb * _BB, _BB)])

        fetch(0, 0, rows_a, sem_a)

        @pl.loop(0, items_per_w // 2)
        def _(p):
            i0 = 2 * p
            fetch(i0 + 1, 1, rows_b, sem_b)
            emit(i0, 0, rows_a, sem_a)

            @pl.when(i0 + 2 < items_per_w)
            def _():
                fetch(i0 + 2, 0, rows_a, sem_a)

            emit(i0 + 1, 1, rows_b, sem_b)

    out_t = gather_kernel(weight, idx_t)
    return out_t.transpose(2, 0, 1)


# speedup vs baseline: 1.2073x; 1.1009x over previous
import jax
import jax.numpy as jnp
from jax import lax
from jax.experimental import pallas as pl
from jax.experimental.pallas import tpu as pltpu
from jax.experimental.pallas import tpu_sc as plsc

_BB = 128


def kernel(input_ids, weight):
    batch, seq = input_ids.shape
    emb_dim = weight.shape[1]
    idx_t = input_ids.T.astype(jnp.int32)

    info = plsc.get_sparse_core_info()
    nw = info.num_cores * info.num_subcores
    assert batch // nw == _BB

    mesh = plsc.VectorSubcoreMesh(
        core_axis_name="core", subcore_axis_name="subcore"
    )

    @pl.kernel(
        out_type=jax.ShapeDtypeStruct((seq, emb_dim, batch), weight.dtype),
        mesh=mesh,
        scratch_types=[
            pltpu.VMEM((seq, _BB), jnp.int32),
            pltpu.VMEM((2, _BB, emb_dim), weight.dtype),
            pltpu.VMEM((2, emb_dim, _BB), weight.dtype),
            pltpu.SemaphoreType.DMA,
            pltpu.SemaphoreType.DMA,
            pltpu.SemaphoreType.DMA,
            pltpu.SemaphoreType.DMA,
        ],
        compiler_params=pltpu.CompilerParams(
            use_tc_tiling_on_sc=False, needs_layout_passes=False
        ),
    )
    def gather_kernel(
        table_hbm, idx_hbm, out_hbm, idx_v, rows, rows_t, sg0, sg1, so0, so1
    ):
        wid = lax.axis_index("subcore") * info.num_cores + lax.axis_index(
            "core"
        )
        b0 = wid * _BB
        sgs = (sg0, sg1)
        sos = (so0, so1)

        pltpu.sync_copy(idx_hbm.at[:, pl.ds(b0, _BB)], idx_v)

        def gather(s, b):
            return pltpu.async_copy(
                table_hbm.at[idx_v.at[s]], rows.at[b], sgs[b]
            )

        def out_copy(s, b):
            return pltpu.make_async_copy(
                rows_t.at[b], out_hbm.at[s, :, pl.ds(b0, _BB)], sos[b]
            )

        gather(0, 0)
        gather(1, 1)

        @pl.loop(0, seq // 2)
        def _(p):
            s0 = 2 * p
            for b in range(2):
                s = s0 + b

                @pl.when(p > 0)
                def _():
                    out_copy(s - 2, b).wait()

                pltpu.make_async_copy(
                    table_hbm.at[idx_v.at[s]], rows.at[b], sgs[b]
                ).wait()
                iota16 = lax.iota(jnp.int32, 16)
                for c in range(emb_dim):
                    cvec = jnp.full((16,), c, dtype=jnp.int32)
                    for k in range(_BB // 16):
                        vals = plsc.load_gather(
                            rows.at[b], [iota16 + k * 16, cvec]
                        )
                        rows_t[b, c, pl.ds(k * 16, 16)] = vals

                @pl.when(p < seq // 2 - 1)
                def _():
                    gather(s + 2, b)

                out_copy(s, b).start()

        out_copy(seq - 2, 0).wait()
        out_copy(seq - 1, 1).wait()

    out_t = gather_kernel(weight, idx_t)
    return out_t.transpose(2, 0, 1)


# speedup vs baseline: 1.4178x; 1.1744x over previous
import jax
import jax.numpy as jnp
from jax import lax
from jax.experimental import pallas as pl
from jax.experimental.pallas import tpu as pltpu
from jax.experimental.pallas import tpu_sc as plsc

_BB = 128


def kernel(input_ids, weight):
    batch, seq = input_ids.shape
    emb_dim = weight.shape[1]
    idx_t = input_ids.T.astype(jnp.int32)

    info = plsc.get_sparse_core_info()
    nw = info.num_cores * info.num_subcores
    assert batch // nw == _BB

    mesh = plsc.VectorSubcoreMesh(
        core_axis_name="core", subcore_axis_name="subcore"
    )

    @pl.kernel(
        out_type=jax.ShapeDtypeStruct((seq, emb_dim, batch), weight.dtype),
        mesh=mesh,
        scratch_types=[
            pltpu.VMEM((seq, _BB), jnp.int32),
            pltpu.VMEM((2, _BB, emb_dim), weight.dtype),
            pltpu.VMEM((2, emb_dim, _BB), weight.dtype),
            pltpu.SemaphoreType.DMA,
            pltpu.SemaphoreType.DMA,
            pltpu.SemaphoreType.DMA,
            pltpu.SemaphoreType.DMA,
        ],
        compiler_params=pltpu.CompilerParams(
            use_tc_tiling_on_sc=False, needs_layout_passes=False
        ),
    )
    def gather_kernel(
        table_hbm, idx_hbm, out_hbm, idx_v, rows, rows_t, sg0, sg1, so0, so1
    ):
        wid = lax.axis_index("subcore") * info.num_cores + lax.axis_index(
            "core"
        )
        b0 = wid * _BB
        sgs = (sg0, sg1)
        sos = (so0, so1)

        pltpu.sync_copy(idx_hbm.at[:, pl.ds(b0, _BB)], idx_v)

        def gather(s, b):
            return pltpu.async_copy(
                table_hbm.at[idx_v.at[s]], rows.at[b], sgs[b]
            )

        def out_copy(s, b):
            return pltpu.make_async_copy(
                rows_t.at[b], out_hbm.at[s, :, pl.ds(b0, _BB)], sos[b]
            )

        gather(0, 0)
        gather(1, 1)

        @pl.loop(0, seq // 2)
        def _(p):
            s0 = 2 * p
            for b in range(2):
                s = s0 + b

                @pl.when(p > 0)
                def _():
                    out_copy(s - 2, b).wait()

                pltpu.make_async_copy(
                    table_hbm.at[idx_v.at[s]], rows.at[b], sgs[b]
                ).wait()
                iota16 = lax.iota(jnp.int32, 16)
                for c in range(emb_dim):
                    cvec = jnp.full((16,), c, dtype=jnp.int32)
                    vals = [
                        plsc.load_gather(rows.at[b], [iota16 + k * 16, cvec])
                        for k in range(_BB // 16)
                    ]
                    for k in range(_BB // 16):
                        rows_t[b, c, pl.ds(k * 16, 16)] = vals[k]

                @pl.when(p < seq // 2 - 1)
                def _():
                    gather(s + 2, b)

                out_copy(s, b).start()

        out_copy(seq - 2, 0).wait()
        out_copy(seq - 1, 1).wait()

    out_t = gather_kernel(weight, idx_t)
    return out_t.transpose(2, 0, 1)


# speedup vs baseline: 1.9954x; 1.4074x over previous
import jax
import jax.numpy as jnp
from jax import lax
from jax.experimental import pallas as pl
from jax.experimental.pallas import tpu as pltpu
from jax.experimental.pallas import tpu_sc as plsc

_BB = 128


def kernel(input_ids, weight):
    batch, seq = input_ids.shape
    emb_dim = weight.shape[1]
    idx_t = input_ids.T.astype(jnp.int32)

    info = plsc.get_sparse_core_info()
    nw = info.num_cores * info.num_subcores
    assert batch // nw == _BB

    mesh = plsc.VectorSubcoreMesh(
        core_axis_name="core", subcore_axis_name="subcore"
    )

    @pl.kernel(
        out_type=jax.ShapeDtypeStruct((seq, emb_dim, batch), weight.dtype),
        mesh=mesh,
        scratch_types=[
            pltpu.VMEM((seq, _BB), jnp.int32),
            pltpu.VMEM((2, _BB, emb_dim), weight.dtype),
            pltpu.VMEM((2, emb_dim, _BB + 1), weight.dtype),
            pltpu.SemaphoreType.DMA,
            pltpu.SemaphoreType.DMA,
            pltpu.SemaphoreType.DMA,
            pltpu.SemaphoreType.DMA,
        ],
        compiler_params=pltpu.CompilerParams(
            use_tc_tiling_on_sc=False, needs_layout_passes=False
        ),
    )
    def gather_kernel(
        table_hbm, idx_hbm, out_hbm, idx_v, rows, rows_t, sg0, sg1, so0, so1
    ):
        wid = lax.axis_index("subcore") * info.num_cores + lax.axis_index(
            "core"
        )
        b0 = wid * _BB
        sgs = (sg0, sg1)
        sos = (so0, so1)

        pltpu.sync_copy(idx_hbm.at[:, pl.ds(b0, _BB)], idx_v)

        def gather(s, b):
            return pltpu.async_copy(
                table_hbm.at[idx_v.at[s]], rows.at[b], sgs[b]
            )

        def out_copy(s, b):
            return pltpu.make_async_copy(
                rows_t.at[b, :, pl.ds(0, _BB)],
                out_hbm.at[s, :, pl.ds(b0, _BB)],
                sos[b],
            )

        gather(0, 0)
        gather(1, 1)

        @pl.loop(0, seq // 2)
        def _(p):
            s0 = 2 * p
            for b in range(2):
                s = s0 + b

                @pl.when(p > 0)
                def _():
                    out_copy(s - 2, b).wait()

                pltpu.make_async_copy(
                    table_hbm.at[idx_v.at[s]], rows.at[b], sgs[b]
                ).wait()
                iota16 = lax.iota(jnp.int32, 16)
                half = emb_dim // 2
                for j0 in range(0, _BB, 8):
                    vals = [
                        (
                            rows.at[b].at[j][pl.ds(0, half)],
                            rows.at[b].at[j][pl.ds(half, half)],
                        )
                        for j in range(j0, j0 + 8)
                    ]
                    for j, (v0, v1) in zip(range(j0, j0 + 8), vals):
                        jvec = jnp.full((16,), j, dtype=jnp.int32)
                        plsc.store_scatter(rows_t.at[b], [iota16, jvec], v0)
                        plsc.store_scatter(
                            rows_t.at[b], [iota16 + half, jvec], v1
                        )

                @pl.when(p < seq // 2 - 1)
                def _():
                    gather(s + 2, b)

                out_copy(s, b).start()

        out_copy(seq - 2, 0).wait()
        out_copy(seq - 1, 1).wait()

    out_t = gather_kernel(weight, idx_t)
    return out_t.transpose(2, 0, 1)


# speedup vs baseline: 2.2353x; 1.1202x over previous
import jax
import jax.numpy as jnp
from jax import lax
from jax.experimental import pallas as pl
from jax.experimental.pallas import tpu as pltpu
from jax.experimental.pallas import tpu_sc as plsc

_BB = 128


def kernel(input_ids, weight):
    batch, seq = input_ids.shape
    emb_dim = weight.shape[1]
    idx_t = input_ids.T.astype(jnp.int32)

    info = plsc.get_sparse_core_info()
    nw = info.num_cores * info.num_subcores
    assert batch // nw == _BB

    mesh = plsc.VectorSubcoreMesh(
        core_axis_name="core", subcore_axis_name="subcore"
    )

    n_cg = emb_dim // 8

    @pl.kernel(
        out_type=jax.ShapeDtypeStruct(
            (seq, n_cg, nw, 8, _BB), weight.dtype
        ),
        mesh=mesh,
        scratch_types=[
            pltpu.VMEM((seq, _BB), jnp.int32),
            pltpu.VMEM((2, _BB, emb_dim), weight.dtype),
            pltpu.VMEM((2, emb_dim, _BB + 1), weight.dtype),
            pltpu.SemaphoreType.DMA,
            pltpu.SemaphoreType.DMA,
            pltpu.SemaphoreType.DMA,
            pltpu.SemaphoreType.DMA,
        ],
        compiler_params=pltpu.CompilerParams(
            use_tc_tiling_on_sc=False, needs_layout_passes=False
        ),
    )
    def gather_kernel(
        table_hbm, idx_hbm, out_hbm, idx_v, rows, rows_t, sg0, sg1, so0, so1
    ):
        wid = lax.axis_index("subcore") * info.num_cores + lax.axis_index(
            "core"
        )
        b0 = wid * _BB
        sgs = (sg0, sg1)
        sos = (so0, so1)

        pltpu.sync_copy(idx_hbm.at[:, pl.ds(b0, _BB)], idx_v)

        def gather(s, b):
            return pltpu.async_copy(
                table_hbm.at[idx_v.at[s]], rows.at[b], sgs[b]
            )

        def out_descs(s, b):
            return [
                pltpu.make_async_copy(
                    rows_t.at[b, pl.ds(cg * 8, 8), pl.ds(0, _BB)],
                    out_hbm.at[s, cg, wid],
                    sos[b],
                )
                for cg in range(n_cg)
            ]

        gather(0, 0)
        gather(1, 1)

        @pl.loop(0, seq // 2)
        def _(p):
            s0 = 2 * p
            for b in range(2):
                s = s0 + b

                @pl.when(p > 0)
                def _():
                    for d in out_descs(s - 2, b):
                        d.wait()

                pltpu.make_async_copy(
                    table_hbm.at[idx_v.at[s]], rows.at[b], sgs[b]
                ).wait()
                iota16 = lax.iota(jnp.int32, 16)
                half = emb_dim // 2
                for j0 in range(0, _BB, 8):
                    vals = [
                        (
                            rows.at[b].at[j][pl.ds(0, half)],
                            rows.at[b].at[j][pl.ds(half, half)],
                        )
                        for j in range(j0, j0 + 8)
                    ]
                    for j, (v0, v1) in zip(range(j0, j0 + 8), vals):
                        jvec = jnp.full((16,), j, dtype=jnp.int32)
                        plsc.store_scatter(rows_t.at[b], [iota16, jvec], v0)
                        plsc.store_scatter(
                            rows_t.at[b], [iota16 + half, jvec], v1
                        )

                @pl.when(p < seq // 2 - 1)
                def _():
                    gather(s + 2, b)

                for d in out_descs(s, b):
                    d.start()

        for d in out_descs(seq - 2, 0):
            d.wait()
        for d in out_descs(seq - 1, 1):
            d.wait()

    out5 = gather_kernel(weight, idx_t)
    return out5.transpose(2, 4, 0, 1, 3).reshape(batch, seq, emb_dim)
